# Initial kernel scaffold; baseline (speedup 1.0000x reference)
import jax
import jax.numpy as jnp
from jax.experimental import pallas as pl


def kernel(input_pos_tensors, table):
    raise NotImplementedError("write your pallas kernel here")


# speedup vs baseline: 2.3151x; 2.3151x over previous
import functools

import jax
import jax.numpy as jnp
from jax import lax
from jax.experimental import pallas as pl
from jax.experimental.pallas import tpu as pltpu
from jax.experimental.pallas import tpu_sc as plsc

MAX_LEN = 8192
EMBED_DIM = 1024
BATCH = 4
SEQ = 8192

_B_TOTAL = BATCH * SEQ
_NC = 2
_NS = 16
_NW = _NC * _NS
_B_PER_W = _B_TOTAL // _NW
_CH = 64
_NCH = _B_PER_W // _CH


@functools.partial(
    pl.kernel,
    out_type=jax.ShapeDtypeStruct((_B_TOTAL, EMBED_DIM), jnp.float32),
    mesh=plsc.VectorSubcoreMesh(core_axis_name="c", subcore_axis_name="s"),
    scratch_types=[
        pltpu.VMEM((_CH,), jnp.int32),
        pltpu.VMEM((_CH, EMBED_DIM), jnp.float32),
        pltpu.SemaphoreType.DMA,
    ],
)
def _gather_rows(idx_hbm, table_hbm, out_hbm, idx_v, rows_v, sem):
    wid = lax.axis_index("s") * _NC + lax.axis_index("c")
    base = wid * _B_PER_W

    def chunk_body(c, _):
        off = base + c * _CH
        pltpu.sync_copy(idx_hbm.at[pl.ds(off, _CH)], idx_v)
        pltpu.async_copy(table_hbm.at[idx_v], rows_v, sem).wait()
        pltpu.sync_copy(rows_v, out_hbm.at[pl.ds(off, _CH)])
        return 0

    lax.fori_loop(0, _NCH, chunk_body, 0)


def kernel(input_pos_tensors, table):
    idx_flat = input_pos_tensors.reshape(-1).astype(jnp.int32)
    out = _gather_rows(idx_flat, table)
    return out.reshape(BATCH, SEQ, EMBED_DIM)


# speedup vs baseline: 2.5653x; 1.1080x over previous
import functools

import jax
import jax.numpy as jnp
from jax import lax
from jax.experimental import pallas as pl
from jax.experimental.pallas import tpu as pltpu
from jax.experimental.pallas import tpu_sc as plsc

MAX_LEN = 8192
EMBED_DIM = 1024
BATCH = 4
SEQ = 8192

_B_TOTAL = BATCH * SEQ
_NC = 2
_NS = 16
_NW = _NC * _NS
_B_PER_W = _B_TOTAL // _NW
_CH = 32
_NCH = _B_PER_W // _CH
_NBUF = 2
_NGRP = _NCH // _NBUF


@functools.partial(
    pl.kernel,
    out_type=jax.ShapeDtypeStruct((_B_TOTAL, EMBED_DIM), jnp.float32),
    mesh=plsc.VectorSubcoreMesh(core_axis_name="c", subcore_axis_name="s"),
    scratch_types=[
        pltpu.VMEM((_NBUF, _CH), jnp.int32),
        pltpu.VMEM((_NBUF, _CH, EMBED_DIM), jnp.float32),
        pltpu.SemaphoreType.DMA,
        pltpu.SemaphoreType.DMA,
        pltpu.SemaphoreType.DMA,
        pltpu.SemaphoreType.DMA,
    ],
)
def _gather_rows(idx_hbm, table_hbm, out_hbm, idx_v, rows_v, g0, g1, w0, w1):
    gsem = (g0, g1)
    wsem = (w0, w1)
    wid = lax.axis_index("s") * _NC + lax.axis_index("c")
    base = wid * _B_PER_W

    def start_gather(c, b):
        off = base + c * _CH
        pltpu.sync_copy(idx_hbm.at[pl.ds(off, _CH)], idx_v.at[b])
        pltpu.async_copy(table_hbm.at[idx_v.at[b]], rows_v.at[b], gsem[b])

    def wait_gather(b):
        pltpu.make_async_copy(
            table_hbm.at[idx_v.at[b]], rows_v.at[b], gsem[b]).wait()

    def start_wb(c, b):
        off = base + c * _CH
        pltpu.async_copy(rows_v.at[b], out_hbm.at[pl.ds(off, _CH)], wsem[b])

    def wait_wb(b):
        pltpu.make_async_copy(
            rows_v.at[b], out_hbm.at[pl.ds(base, _CH)], wsem[b]).wait()

    for b in range(_NBUF):
        start_gather(b, b)

    def group_body(g, _):
        c0 = g * _NBUF
        for b in range(_NBUF):
            c = c0 + b
            wait_gather(b)
            start_wb(c, b)
            wait_wb(b)
            start_gather(c + _NBUF, b)
        return 0

    lax.fori_loop(0, _NGRP - 1, group_body, 0)

    c0 = (_NGRP - 1) * _NBUF
    for b in range(_NBUF):
        wait_gather(b)
        start_wb(c0 + b, b)
    for b in range(_NBUF):
        wait_wb(b)


def kernel(input_pos_tensors, table):
    idx_flat = input_pos_tensors.reshape(-1).astype(jnp.int32)
    out = _gather_rows(idx_flat, table)
    return out.reshape(BATCH, SEQ, EMBED_DIM)
